# baseline (speedup 1.0000x reference)
import functools

import jax
import jax.numpy as jnp
from jax.experimental import pallas as pl
from jax.experimental.pallas import tpu as pltpu

_N = 4096
_BI = 512
_BJ = 1024
_BJS = 1024


def _inc_body(m_ref, out_ref):
    j = pl.program_id(0)

    @pl.when(j == 0)
    def _():
        out_ref[...] = jnp.zeros_like(out_ref)

    out_ref[...] += jnp.sum(m_ref[...], axis=0, keepdims=True)


def _column_sums(M):
    nJ = _N // _BJS
    return pl.pallas_call(
        _inc_body,
        grid=(nJ,),
        in_specs=[pl.BlockSpec((_BJS, _N), lambda j: (j, 0))],
        out_specs=pl.BlockSpec((1, _N), lambda j: (0, 0)),
        out_shape=jax.ShapeDtypeStruct((1, _N), jnp.float32),
        compiler_params=pltpu.CompilerParams(
            dimension_semantics=("arbitrary",),
        ),
    )(M)


def _layer_body(*args, nJ, in_dim, has_ws, apply_relu):
    if has_ws:
        (m_ref, hj_ref, hi_ref, inc_ref, w1_ref, b1_ref, w2_ref, b2_ref,
         ws_ref, bs_ref, g_ref, bt_ref, out_ref,
         tx_s, res_s, acc_s, s_s) = args
    else:
        (m_ref, hj_ref, hi_ref, inc_ref, w1_ref, b1_ref, w2_ref, b2_ref,
         g_ref, bt_ref, out_ref,
         tx_s, acc_s, s_s) = args
        ws_ref = bs_ref = res_s = None

    i = pl.program_id(0)
    j = pl.program_id(1)

    @pl.when(i == 0)
    def _():
        hj = hj_ref[...]
        tx = jnp.dot(hj, w1_ref[...], preferred_element_type=jnp.float32,
                     precision=jax.lax.Precision.HIGHEST) + b1_ref[...]
        tx_s[pl.ds(j * _BJ, _BJ), :] = tx
        if has_ws:
            res_s[pl.ds(j * _BJ, _BJ), :] = (
                jnp.dot(hj, ws_ref[...], preferred_element_type=jnp.float32,
                        precision=jax.lax.Precision.HIGHEST) + bs_ref[...])

    @pl.when(j == 0)
    def _():
        acc_s[...] = jnp.zeros_like(acc_s)
        s_s[...] = jnp.zeros_like(s_s)

    inc_row = inc_ref[...]
    thresh = 1e-6 * (inc_row + 1e-8)
    m = m_ref[...]
    mm = jnp.where(m > thresh, m, 0.0)
    s_s[...] += jnp.sum(mm, axis=0, keepdims=True)
    mm_bf = mm.astype(jnp.bfloat16)
    tx_bf = tx_s[pl.ds(j * _BJ, _BJ), :].astype(jnp.bfloat16)
    acc_s[...] += jax.lax.dot_general(
        mm_bf, tx_bf, (((0,), (0,)), ((), ())),
        preferred_element_type=jnp.float32)

    @pl.when(j == nJ - 1)
    def _():
        s_row = s_s[...]
        denom_row = s_row + 1e-8 * (inc_row + 1e-8)
        packed = jnp.concatenate(
            [s_row, denom_row, jnp.zeros((6, s_row.shape[1]), jnp.float32)],
            axis=0)
        packed_t = packed.T
        s_col = packed_t[:, 0:1]
        denom_col = packed_t[:, 1:2]
        txi = tx_s[pl.ds(i * _BI, _BI), :]
        agg = jnp.where(s_col > 0.0, acc_s[...] / denom_col, txi)
        out = jnp.dot(agg, w2_ref[...], preferred_element_type=jnp.float32,
                      precision=jax.lax.Precision.HIGHEST) + b2_ref[...]
        if has_ws:
            out = out + res_s[pl.ds(i * _BI, _BI), :]
        else:
            out = out + hi_ref[...]
        mu = jnp.mean(out, axis=-1, keepdims=True)
        var = jnp.mean((out - mu) ** 2, axis=-1, keepdims=True)
        out = (out - mu) * jax.lax.rsqrt(var + 1e-5) * g_ref[...] + bt_ref[...]
        if apply_relu:
            out = jnp.maximum(out, 0.0)
        out_ref[...] = out


def _layer(h, M, inc, W1, b1, W2, b2, Ws, bs, g, bt, apply_relu):
    n, in_dim = h.shape
    hdim = W1.shape[1]
    nI = _N // _BI
    nJ = _N // _BJ
    has_ws = Ws is not None

    row = lambda v: v.reshape(1, -1)
    in_specs = [
        pl.BlockSpec((_BJ, _BI), lambda i, j: (j, i)),
        pl.BlockSpec((_BJ, in_dim), lambda i, j: (j, 0)),
        pl.BlockSpec((_BI, in_dim), lambda i, j: (i, 0)),
        pl.BlockSpec((1, _BI), lambda i, j: (0, i)),
        pl.BlockSpec((in_dim, hdim), lambda i, j: (0, 0)),
        pl.BlockSpec((1, hdim), lambda i, j: (0, 0)),
        pl.BlockSpec((hdim, hdim), lambda i, j: (0, 0)),
        pl.BlockSpec((1, hdim), lambda i, j: (0, 0)),
    ]
    inputs = [M, h, h, inc, W1, row(b1), W2, row(b2)]
    if has_ws:
        in_specs += [
            pl.BlockSpec((in_dim, hdim), lambda i, j: (0, 0)),
            pl.BlockSpec((1, hdim), lambda i, j: (0, 0)),
        ]
        inputs += [Ws, row(bs)]
    in_specs += [
        pl.BlockSpec((1, hdim), lambda i, j: (0, 0)),
        pl.BlockSpec((1, hdim), lambda i, j: (0, 0)),
    ]
    inputs += [row(g), row(bt)]

    scratch = [pltpu.VMEM((_N, hdim), jnp.float32)]
    if has_ws:
        scratch.append(pltpu.VMEM((_N, hdim), jnp.float32))
    scratch += [
        pltpu.VMEM((_BI, hdim), jnp.float32),
        pltpu.VMEM((1, _BI), jnp.float32),
    ]

    body = functools.partial(_layer_body, nJ=nJ, in_dim=in_dim,
                             has_ws=has_ws, apply_relu=apply_relu)
    return pl.pallas_call(
        body,
        grid=(nI, nJ),
        in_specs=in_specs,
        out_specs=pl.BlockSpec((_BI, hdim), lambda i, j: (i, 0)),
        out_shape=jax.ShapeDtypeStruct((n, hdim), jnp.float32),
        scratch_shapes=scratch,
        compiler_params=pltpu.CompilerParams(
            dimension_semantics=("arbitrary", "arbitrary"),
        ),
    )(*inputs)


def kernel(node_features, mobility_matrix, W1_0, b1_0, W2_0, b2_0, Ws_0,
           bs_0, g_0, bt_0, W1_1, b1_1, W2_1, b2_1, g_1, bt_1):
    inc = _column_sums(mobility_matrix)
    h = _layer(node_features, mobility_matrix, inc,
               W1_0, b1_0, W2_0, b2_0, Ws_0, bs_0, g_0, bt_0,
               apply_relu=False)
    out = _layer(h, mobility_matrix, inc,
                 W1_1, b1_1, W2_1, b2_1, None, None, g_1, bt_1,
                 apply_relu=True)
    return out
